# baseline (speedup 1.0000x reference)
import functools

import jax
import jax.numpy as jnp
from jax.experimental import pallas as pl
from jax.experimental.pallas import tpu as pltpu

N = 100000
E = 6400000
H = 184
LANES = 256
B = 2000
NB = N // B


def _dense_body(cen_ref, acc_ref, x4_ref, wx_ref, wn_ref, bg_ref,
                xc_ref, w2_ref, b2_ref, w21_ref, b21_ref,
                w1a_ref, w1b_ref, b1_ref, w4_ref, b4_ref,
                o_ref, zmax_s, hc_s):
    b = pl.program_id(0)

    @pl.when(b == 0)
    def _():
        zmax_s[...] = jnp.full_like(zmax_s[...], -jnp.inf)
        hc_s[...] = jnp.zeros_like(hc_s[...])

    acc = acc_ref[0] + acc_ref[1]
    cnt = acc[:, 4:5]
    rc = 1.0 / jnp.maximum(cnt, 1.0)
    mean8 = acc * rc
    z = (jnp.dot(x4_ref[...], wx_ref[...], preferred_element_type=jnp.float32)
         + jnp.dot(mean8, wn_ref[...], preferred_element_type=jnp.float32)
         + bg_ref[...])

    zmax_s[...] = jnp.maximum(zmax_s[...],
                              jnp.max(z.reshape(-1, 8, LANES), axis=0))

    gid = b * B + jax.lax.broadcasted_iota(jnp.int32, (B, 1), 0)
    m = gid == cen_ref[0, 0]
    hc_s[...] += jnp.sum(jnp.where(m, z, 0.0), axis=0, keepdims=True)

    @pl.when(b == NB - 1)
    def _():
        relu = lambda v: jnp.maximum(v, 0.0)
        x1 = relu(jnp.max(zmax_s[...], axis=0, keepdims=True))
        hcen = relu(hc_s[...])
        g = hcen - x1
        dot = lambda a, w: jnp.dot(a, w[...], preferred_element_type=jnp.float32)
        md = relu(dot(xc_ref[...], w2_ref) + b2_ref[...])
        md = relu(dot(md, w21_ref) + b21_ref[...])
        z1 = relu(dot(g, w1a_ref) + dot(md, w1b_ref) + b1_ref[...])
        o_ref[...] = dot(z1, w4_ref) + b4_ref[...]


def _pad2(a, rows, cols):
    return jnp.pad(a, ((0, rows - a.shape[0]), (0, cols - a.shape[1])))


def _dense_head(acc2, x4p, center_idx, x, W_root, W_nbr, b_g,
                W2, b2, W2_1, b2_1, W1, b1, W4, b4):
    wx = _pad2(W_root.T, 8, LANES)
    wn = _pad2(W_nbr.T, 8, LANES)
    bg = _pad2(b_g[None, :], 1, LANES)
    xc = jax.lax.dynamic_slice_in_dim(x, center_idx, 1, 0)[:, 4:6]
    xc = _pad2(xc, 1, LANES)
    w2 = _pad2(W2.T, LANES, LANES)
    b2p = _pad2(b2[None, :], 1, LANES)
    w21 = _pad2(W2_1.T, LANES, LANES)
    b21 = _pad2(b2_1[None, :], 1, LANES)
    w1t = W1.T
    w1a = _pad2(w1t[:H], LANES, LANES)
    w1b = _pad2(w1t[H:], LANES, LANES)
    b1p = _pad2(b1[None, :], 1, LANES)
    w4 = _pad2(W4.T, LANES, 128)
    b4p = _pad2(b4[None, :], 1, 128)
    cen = jnp.asarray(center_idx, jnp.int32).reshape(1, 1)

    full = lambda shape: pl.BlockSpec(shape, lambda b: tuple(0 for _ in shape))
    grid_specs = dict(
        grid=(NB,),
        in_specs=[
            pl.BlockSpec(memory_space=pltpu.SMEM),
            pl.BlockSpec((2, B, 8), lambda b: (0, b, 0)),
            pl.BlockSpec((B, 8), lambda b: (b, 0)),
            full((8, LANES)), full((8, LANES)), full((1, LANES)),
            full((1, LANES)),
            full((LANES, LANES)), full((1, LANES)),
            full((LANES, LANES)), full((1, LANES)),
            full((LANES, LANES)), full((LANES, LANES)), full((1, LANES)),
            full((LANES, 128)), full((1, 128)),
        ],
        out_specs=pl.BlockSpec((1, 128), lambda b: (0, 0)),
        scratch_shapes=[pltpu.VMEM((8, LANES), jnp.float32),
                        pltpu.VMEM((1, LANES), jnp.float32)],
    )
    out = pl.pallas_call(
        _dense_body,
        out_shape=jax.ShapeDtypeStruct((1, 128), jnp.float32),
        **grid_specs,
    )(cen, acc2, x4p, wx, wn, bg, xc, w2, b2p, w21, b21, w1a, w1b, b1p, w4, b4p)
    return out[:, :5]


def kernel(x, edge_index, edge_weight, center_idx, W_root, W_nbr, b_g,
           W2, b2, W2_1, b2_1, W1, b1, W4, b4):
    x4 = x[:, :4]
    x4p = jnp.pad(x4, ((0, 0), (0, 4)))

    src = edge_index[0]
    dst = edge_index[1]
    msgs = x4[src] * edge_weight[:, None]
    summed = jax.ops.segment_sum(msgs, dst, num_segments=N)
    counts = jax.ops.segment_sum(jnp.ones((E,), jnp.float32), dst, num_segments=N)
    acc = jnp.concatenate([summed, counts[:, None], jnp.zeros((N, 3), jnp.float32)], axis=1)
    acc2 = jnp.stack([acc, jnp.zeros_like(acc)])

    return _dense_head(acc2, x4p, center_idx, x, W_root, W_nbr, b_g,
                       W2, b2, W2_1, b2_1, W1, b1, W4, b4)


# speedup vs baseline: 41.9426x; 41.9426x over previous
import dataclasses

import jax
import jax.numpy as jnp
from jax import lax
from jax.experimental import pallas as pl
from jax.experimental.pallas import tpu as pltpu
from jax.experimental.pallas import tpu_sc as plsc

N = 100000
E = 6400000
H = 184
ACC_N = 100096
G = E // 128
GPW = 1560
GPC = 8
NCH = GPW // GPC
TAILW = 10
TAILG = 8

LANES = 256
B = 2000
NB = N // B

_mesh = plsc.VectorSubcoreMesh(core_axis_name="c", subcore_axis_name="s")


def _seg_kernel(x8_hbm, src_hbm, dst_hbm, w_hbm, z_hbm, out_hbm,
                x8_sh, acc_sh, src_v, dst_v, w_v, rows_a, rows_b,
                g_a, g_b, sc_a, sc_b):
    c = lax.axis_index("c")
    s = lax.axis_index("s")
    wid = c * 16 + s

    pltpu.sync_copy(x8_hbm.at[pl.ds(s * (ACC_N // 16), ACC_N // 16)],
                    x8_sh.at[pl.ds(s * (ACC_N // 16), ACC_N // 16)])
    pltpu.sync_copy(z_hbm.at[pl.ds(s * (ACC_N // 16), ACC_N // 16)],
                    acc_sh.at[pl.ds(s * (ACC_N // 16), ACC_N // 16)])
    plsc.subcore_barrier()

    iota = lax.iota(jnp.int32, 16)

    def scale(buf, k):
        for sub in range(8):
            wv = w_v[k, pl.ds(sub * 16, 16)]
            ridx = iota + (16 * sub)
            for f in range(4):
                cidx = jnp.full((16,), f, jnp.int32)
                vals = plsc.load_gather(buf, [ridx, cidx])
                plsc.store_scatter(buf, [ridx, cidx], vals * wv)

    def do_pairs(npairs):
        for j in range(npairs):
            k0 = 2 * j
            k1 = 2 * j + 1
            if j > 0:
                pltpu.make_async_copy(rows_a, acc_sh.at[dst_v.at[k0 - 2]], sc_a).wait()
                pltpu.make_async_copy(rows_b, acc_sh.at[dst_v.at[k1 - 2]], sc_b).wait()
            ha = pltpu.async_copy(x8_sh.at[src_v.at[k0]], rows_a, g_a)
            hb = pltpu.async_copy(x8_sh.at[src_v.at[k1]], rows_b, g_b)
            ha.wait()
            scale(rows_a, k0)
            pltpu.async_copy(rows_a, acc_sh.at[dst_v.at[k0]], sc_a, add=True)
            hb.wait()
            scale(rows_b, k1)
            pltpu.async_copy(rows_b, acc_sh.at[dst_v.at[k1]], sc_b, add=True)
        pltpu.make_async_copy(rows_a, acc_sh.at[dst_v.at[2 * npairs - 2]], sc_a).wait()
        pltpu.make_async_copy(rows_b, acc_sh.at[dst_v.at[2 * npairs - 1]], sc_b).wait()

    gs = wid * GPW

    @pl.loop(0, NCH)
    def _chunks(ci):
        r0 = gs + ci * GPC
        pltpu.sync_copy(src_hbm.at[pl.ds(r0, GPC)], src_v)
        pltpu.sync_copy(dst_hbm.at[pl.ds(r0, GPC)], dst_v)
        pltpu.sync_copy(w_hbm.at[pl.ds(r0, GPC)], w_v)
        do_pairs(GPC // 2)

    @pl.when(wid < TAILW)
    def _():
        r1 = 32 * GPW + wid * TAILG
        pltpu.sync_copy(src_hbm.at[pl.ds(r1, TAILG)], src_v.at[pl.ds(0, TAILG)])
        pltpu.sync_copy(dst_hbm.at[pl.ds(r1, TAILG)], dst_v.at[pl.ds(0, TAILG)])
        pltpu.sync_copy(w_hbm.at[pl.ds(r1, TAILG)], w_v.at[pl.ds(0, TAILG)])
        do_pairs(TAILG // 2)

    plsc.subcore_barrier()
    pltpu.sync_copy(acc_sh.at[pl.ds(s * (ACC_N // 16), ACC_N // 16)],
                    out_hbm.at[c].at[pl.ds(s * (ACC_N // 16), ACC_N // 16)])


_sc_params = pltpu.CompilerParams()
if "needs_layout_passes" in pltpu.CompilerParams.__dataclass_fields__:
    _sc_params = dataclasses.replace(_sc_params, needs_layout_passes=False)
if "use_tc_tiling_on_sc" in pltpu.CompilerParams.__dataclass_fields__:
    _sc_params = dataclasses.replace(_sc_params, use_tc_tiling_on_sc=False)


def _segment_accumulate(x8, src2, dst2, w2, zeros):
    k = pl.kernel(
        _seg_kernel,
        out_type=jax.ShapeDtypeStruct((2, ACC_N, 8), jnp.float32),
        mesh=_mesh,
        compiler_params=_sc_params,
        scratch_types=[
            pltpu.VMEM_SHARED((ACC_N, 8), jnp.float32),
            pltpu.VMEM_SHARED((ACC_N, 8), jnp.float32),
            pltpu.VMEM((GPC, 128), jnp.int32),
            pltpu.VMEM((GPC, 128), jnp.int32),
            pltpu.VMEM((GPC, 128), jnp.float32),
            pltpu.VMEM((128, 8), jnp.float32),
            pltpu.VMEM((128, 8), jnp.float32),
            pltpu.SemaphoreType.DMA,
            pltpu.SemaphoreType.DMA,
            pltpu.SemaphoreType.DMA,
            pltpu.SemaphoreType.DMA,
        ],
    )
    return k(x8, src2, dst2, w2, zeros)


def _dense_body(cen_ref, acc_ref, x4_ref, wx_ref, wn_ref, bg_ref,
                xc_ref, w2_ref, b2_ref, w21_ref, b21_ref,
                w1a_ref, w1b_ref, b1_ref, w4_ref, b4_ref,
                o_ref, zmax_s, hc_s):
    b = pl.program_id(0)

    @pl.when(b == 0)
    def _():
        zmax_s[...] = jnp.full_like(zmax_s[...], -jnp.inf)
        hc_s[...] = jnp.zeros_like(hc_s[...])

    acc = acc_ref[0] + acc_ref[1]
    cnt = acc[:, 4:5]
    rc = 1.0 / jnp.maximum(cnt, 1.0)
    mean8 = acc * rc
    z = (jnp.dot(x4_ref[...], wx_ref[...], preferred_element_type=jnp.float32)
         + jnp.dot(mean8, wn_ref[...], preferred_element_type=jnp.float32)
         + bg_ref[...])

    zmax_s[...] = jnp.maximum(zmax_s[...],
                              jnp.max(z.reshape(-1, 8, LANES), axis=0))

    gid = b * B + jax.lax.broadcasted_iota(jnp.int32, (B, 1), 0)
    m = gid == cen_ref[0, 0]
    hc_s[...] += jnp.sum(jnp.where(m, z, 0.0), axis=0, keepdims=True)

    @pl.when(b == NB - 1)
    def _():
        relu = lambda v: jnp.maximum(v, 0.0)
        x1 = relu(jnp.max(zmax_s[...], axis=0, keepdims=True))
        hcen = relu(hc_s[...])
        g = hcen - x1
        dot = lambda a, w: jnp.dot(a, w[...], preferred_element_type=jnp.float32)
        md = relu(dot(xc_ref[...], w2_ref) + b2_ref[...])
        md = relu(dot(md, w21_ref) + b21_ref[...])
        z1 = relu(dot(g, w1a_ref) + dot(md, w1b_ref) + b1_ref[...])
        o_ref[...] = dot(z1, w4_ref) + b4_ref[...]


def _pad2(a, rows, cols):
    return jnp.pad(a, ((0, rows - a.shape[0]), (0, cols - a.shape[1])))


def _dense_head(acc2, x4p, center_idx, x, W_root, W_nbr, b_g,
                W2, b2, W2_1, b2_1, W1, b1, W4, b4):
    wx = _pad2(W_root.T, 8, LANES)
    wn = _pad2(W_nbr.T, 8, LANES)
    bg = _pad2(b_g[None, :], 1, LANES)
    xc = jax.lax.dynamic_slice_in_dim(x, center_idx, 1, 0)[:, 4:6]
    xc = _pad2(xc, 1, LANES)
    w2 = _pad2(W2.T, LANES, LANES)
    b2p = _pad2(b2[None, :], 1, LANES)
    w21 = _pad2(W2_1.T, LANES, LANES)
    b21 = _pad2(b2_1[None, :], 1, LANES)
    w1t = W1.T
    w1a = _pad2(w1t[:H], LANES, LANES)
    w1b = _pad2(w1t[H:], LANES, LANES)
    b1p = _pad2(b1[None, :], 1, LANES)
    w4 = _pad2(W4.T, LANES, 128)
    b4p = _pad2(b4[None, :], 1, 128)
    cen = jnp.asarray(center_idx, jnp.int32).reshape(1, 1)

    full = lambda shape: pl.BlockSpec(shape, lambda b: tuple(0 for _ in shape))
    grid_specs = dict(
        grid=(NB,),
        in_specs=[
            pl.BlockSpec(memory_space=pltpu.SMEM),
            pl.BlockSpec((2, B, 8), lambda b: (0, b, 0)),
            pl.BlockSpec((B, 8), lambda b: (b, 0)),
            full((8, LANES)), full((8, LANES)), full((1, LANES)),
            full((1, LANES)),
            full((LANES, LANES)), full((1, LANES)),
            full((LANES, LANES)), full((1, LANES)),
            full((LANES, LANES)), full((LANES, LANES)), full((1, LANES)),
            full((LANES, 128)), full((1, 128)),
        ],
        out_specs=pl.BlockSpec((1, 128), lambda b: (0, 0)),
        scratch_shapes=[pltpu.VMEM((8, LANES), jnp.float32),
                        pltpu.VMEM((1, LANES), jnp.float32)],
    )
    out = pl.pallas_call(
        _dense_body,
        out_shape=jax.ShapeDtypeStruct((1, 128), jnp.float32),
        **grid_specs,
    )(cen, acc2, x4p, wx, wn, bg, xc, w2, b2p, w21, b21, w1a, w1b, b1p, w4, b4p)
    return out[:, :5]


def kernel(x, edge_index, edge_weight, center_idx, W_root, W_nbr, b_g,
           W2, b2, W2_1, b2_1, W1, b1, W4, b4):
    x4 = x[:, :4]
    x4p = jnp.pad(x4, ((0, 0), (0, 4)))
    x8 = jnp.concatenate(
        [x4, jnp.ones((N, 1), jnp.float32), jnp.zeros((N, 3), jnp.float32)],
        axis=1)
    x8 = jnp.pad(x8, ((0, ACC_N - N), (0, 0)))
    src2 = edge_index[0].reshape(G, 128)
    dst2 = edge_index[1].reshape(G, 128)
    w2 = edge_weight.reshape(G, 128)
    zeros = jnp.zeros((ACC_N, 8), jnp.float32)

    acc2 = _segment_accumulate(x8, src2, dst2, w2, zeros)

    return _dense_head(acc2, x4p, center_idx, x, W_root, W_nbr, b_g,
                       W2, b2, W2_1, b2_1, W1, b1, W4, b4)
